# baseline (device time: 49717 ns/iter reference)
import jax
import jax.numpy as jnp
from jax import lax
from jax.experimental import pallas as pl
from jax.experimental.pallas import tpu as pltpu

N_DEV = 8
N_LAYERS = 3
CHUNK = 64

L = (0, 1, 3, 2, 4, 5, 7, 6)
MASKS_DESC = (6, 2, 5, 7, 1, 3, 4)
MASKS_ASC = (1, 3, 4, 2, 5, 7, 6)


def kernel(x, Win0, Wout0, Win1, Wout1, Win2, Wout2):
    b, d_sh = x.shape
    _, h_dim = Win0.shape
    _, o_sh = Wout0.shape

    def body(x_ref, win0_ref, wout0_ref, win1_ref, wout1_ref, win2_ref,
             wout2_ref, out_ref, h_ref, rs_rbuf, hg_ref,
             rs_send_sems, rs_recv_sems, ag_send_sems, ag_recv_sems):
        my_p = lax.axis_index("i")
        p0 = jnp.bitwise_and(my_p, 1)
        p1 = jnp.bitwise_and(my_p >> 1, 1)
        p2 = jnp.bitwise_and(my_p >> 2, 1)
        my_q = 4 * p2 + 2 * p1 + jnp.bitwise_xor(p0, p1)
        my_row = my_q * CHUNK

        wins = (win0_ref, win1_ref, win2_ref)
        wouts = (wout0_ref, wout1_ref, wout2_ref)

        def rs_send(layer, j, c):
            rdma = pltpu.make_async_remote_copy(
                src_ref=h_ref.at[pl.ds(c * CHUNK, CHUNK)],
                dst_ref=rs_rbuf.at[my_p],
                send_sem=rs_send_sems.at[layer * N_DEV + j],
                recv_sem=rs_recv_sems.at[layer * N_DEV + my_p],
                device_id=(j,),
                device_id_type=pl.DeviceIdType.MESH,
            )
            rdma.start()

        def rs_desc(layer, j):
            return pltpu.make_async_remote_copy(
                src_ref=rs_rbuf.at[j],
                dst_ref=rs_rbuf.at[j],
                send_sem=rs_send_sems.at[layer * N_DEV + j],
                recv_sem=rs_recv_sems.at[layer * N_DEV + j],
                device_id=(j,),
                device_id_type=pl.DeviceIdType.MESH,
            )

        def ag_send(layer, j):
            rdma = pltpu.make_async_remote_copy(
                src_ref=hg_ref.at[pl.ds(my_row, CHUNK)],
                dst_ref=hg_ref.at[pl.ds(my_row, CHUNK)],
                send_sem=ag_send_sems.at[layer * N_DEV + j],
                recv_sem=ag_recv_sems.at[layer * N_DEV + my_p],
                device_id=(j,),
                device_id_type=pl.DeviceIdType.MESH,
            )
            rdma.start()

        def ag_desc(layer, j, c):
            return pltpu.make_async_remote_copy(
                src_ref=hg_ref.at[pl.ds(c * CHUNK, CHUNK)],
                dst_ref=hg_ref.at[pl.ds(c * CHUNK, CHUNK)],
                send_sem=ag_send_sems.at[layer * N_DEV + j],
                recv_sem=ag_recv_sems.at[layer * N_DEV + j],
                device_id=(j,),
                device_id_type=pl.DeviceIdType.MESH,
            )

        def reduce_relu_broadcast(layer, part_own):
            acc = part_own
            for mask in MASKS_ASC:
                j = jnp.bitwise_xor(my_p, mask)
                rs_desc(layer, j).wait_recv()
                acc = acc + rs_rbuf[j].astype(jnp.float32)
            myh = jnp.maximum(acc, 0.0).astype(jnp.bfloat16)
            if layer > 0:
                for mask in MASKS_ASC:
                    j = jnp.bitwise_xor(my_p, mask)
                    ag_desc(layer - 1, j, 0).wait_send()
            hg_ref[pl.ds(my_row, CHUNK), :] = myh
            for mask in MASKS_DESC:
                ag_send(layer, jnp.bitwise_xor(my_p, mask))
            return myh

        xv = x_ref[...].astype(jnp.bfloat16)
        w_in = wins[0][...].astype(jnp.bfloat16)
        partial = jnp.dot(xv, w_in, preferred_element_type=jnp.float32)
        h_ref[...] = partial.astype(jnp.bfloat16)
        for mask in MASKS_DESC:
            j = jnp.bitwise_xor(my_p, mask)
            c = jnp.bitwise_xor(my_q, L[mask])
            rs_send(0, j, c)
        part_own = h_ref[pl.ds(my_row, CHUNK), :].astype(jnp.float32)
        myh = reduce_relu_broadcast(0, part_own)

        for layer in (1, 2):
            w_out = wouts[layer - 1][...].astype(jnp.bfloat16)
            w_in = wins[layer][...].astype(jnp.bfloat16)

            xv_own = jnp.dot(myh, w_out, preferred_element_type=jnp.float32)
            part_own = jnp.dot(
                xv_own.astype(jnp.bfloat16), w_in,
                preferred_element_type=jnp.float32,
            )

            for mask in MASKS_ASC:
                j = jnp.bitwise_xor(my_p, mask)
                c = jnp.bitwise_xor(my_q, L[mask])
                ag_desc(layer - 1, j, c).wait_recv()
                xv_j = jnp.dot(
                    hg_ref[pl.ds(c * CHUNK, CHUNK), :], w_out,
                    preferred_element_type=jnp.float32,
                ).astype(jnp.bfloat16)
                part_j = jnp.dot(
                    xv_j, w_in, preferred_element_type=jnp.float32
                ).astype(jnp.bfloat16)
                rs_desc(layer - 1, j).wait_send()
                h_ref[pl.ds(c * CHUNK, CHUNK), :] = part_j
                rs_send(layer, j, c)

            myh = reduce_relu_broadcast(layer, part_own)

        w_out = wouts[2][...].astype(jnp.bfloat16)
        out_ref[pl.ds(my_row, CHUNK), :] = jnp.dot(
            myh, w_out, preferred_element_type=jnp.float32
        )
        for mask in MASKS_ASC:
            j = jnp.bitwise_xor(my_p, mask)
            c = jnp.bitwise_xor(my_q, L[mask])
            ag_desc(2, j, c).wait_recv()
            out_ref[pl.ds(c * CHUNK, CHUNK), :] = jnp.dot(
                hg_ref[pl.ds(c * CHUNK, CHUNK), :], w_out,
                preferred_element_type=jnp.float32,
            )

        for mask in MASKS_ASC:
            j = jnp.bitwise_xor(my_p, mask)
            rs_desc(2, j).wait_send()
            ag_desc(2, j, 0).wait_send()

    n_sems = N_LAYERS * N_DEV
    return pl.pallas_call(
        body,
        out_shape=jax.ShapeDtypeStruct((b, o_sh), jnp.float32),
        in_specs=[pl.BlockSpec(memory_space=pltpu.VMEM)] * 7,
        out_specs=pl.BlockSpec(memory_space=pltpu.VMEM),
        scratch_shapes=[
            pltpu.VMEM((b, h_dim), jnp.bfloat16),
            pltpu.VMEM((N_DEV, CHUNK, h_dim), jnp.bfloat16),
            pltpu.VMEM((b, h_dim), jnp.bfloat16),
            pltpu.SemaphoreType.DMA((n_sems,)),
            pltpu.SemaphoreType.DMA((n_sems,)),
            pltpu.SemaphoreType.DMA((n_sems,)),
            pltpu.SemaphoreType.DMA((n_sems,)),
        ],
    )(x, Win0, Wout0, Win1, Wout1, Win2, Wout2)


# device time: 44666 ns/iter; 1.1131x vs baseline; 1.1131x over previous
import jax
import jax.numpy as jnp
from jax import lax
from jax.experimental import pallas as pl
from jax.experimental.pallas import tpu as pltpu

N_DEV = 8
N_LAYERS = 3
CHUNK = 64
HALF = 256

L = (0, 1, 3, 2, 4, 5, 7, 6)
MASKS_DESC = (6, 2, 5, 7, 1, 3, 4)
MASKS_ASC = (1, 3, 4, 2, 5, 7, 6)


def kernel(x, Win0, Wout0, Win1, Wout1, Win2, Wout2):
    b, d_sh = x.shape
    _, h_dim = Win0.shape
    _, o_sh = Wout0.shape

    def body(x_ref, win0_ref, wout0_ref, win1_ref, wout1_ref, win2_ref,
             wout2_ref, out_ref, h_ref, rs_rbuf, hg_ref,
             rs_send_sems, rs_recv_sems, ag_send_sems, ag_recv_sems):
        my_p = lax.axis_index("i")
        p0 = jnp.bitwise_and(my_p, 1)
        p1 = jnp.bitwise_and(my_p >> 1, 1)
        p2 = jnp.bitwise_and(my_p >> 2, 1)
        my_q = 4 * p2 + 2 * p1 + jnp.bitwise_xor(p0, p1)
        my_row = my_q * CHUNK

        wins = (win0_ref, win1_ref, win2_ref)
        wouts = (wout0_ref, wout1_ref, wout2_ref)

        def slot(layer, half, j):
            return (layer * 2 + half) * N_DEV + j

        def rs_send(layer, half, j, c):
            rdma = pltpu.make_async_remote_copy(
                src_ref=h_ref.at[half, pl.ds(c * CHUNK, CHUNK)],
                dst_ref=rs_rbuf.at[half, my_p],
                send_sem=rs_send_sems.at[slot(layer, half, j)],
                recv_sem=rs_recv_sems.at[slot(layer, half, my_p)],
                device_id=(j,),
                device_id_type=pl.DeviceIdType.MESH,
            )
            rdma.start()

        def rs_desc(layer, half, j):
            return pltpu.make_async_remote_copy(
                src_ref=rs_rbuf.at[half, j],
                dst_ref=rs_rbuf.at[half, j],
                send_sem=rs_send_sems.at[slot(layer, half, j)],
                recv_sem=rs_recv_sems.at[slot(layer, half, j)],
                device_id=(j,),
                device_id_type=pl.DeviceIdType.MESH,
            )

        def ag_send(layer, half, j):
            rdma = pltpu.make_async_remote_copy(
                src_ref=hg_ref.at[half, pl.ds(my_row, CHUNK)],
                dst_ref=hg_ref.at[half, pl.ds(my_row, CHUNK)],
                send_sem=ag_send_sems.at[slot(layer, half, j)],
                recv_sem=ag_recv_sems.at[slot(layer, half, my_p)],
                device_id=(j,),
                device_id_type=pl.DeviceIdType.MESH,
            )
            rdma.start()

        def ag_desc(layer, half, j, c):
            return pltpu.make_async_remote_copy(
                src_ref=hg_ref.at[half, pl.ds(c * CHUNK, CHUNK)],
                dst_ref=hg_ref.at[half, pl.ds(c * CHUNK, CHUNK)],
                send_sem=ag_send_sems.at[slot(layer, half, j)],
                recv_sem=ag_recv_sems.at[slot(layer, half, j)],
                device_id=(j,),
                device_id_type=pl.DeviceIdType.MESH,
            )

        xv = x_ref[...].astype(jnp.bfloat16)
        for layer in range(N_LAYERS):
            w_in = wins[layer]
            w_out = wouts[layer]

            for half in range(2):
                p_half = jnp.dot(
                    xv, w_in[:, half * HALF:(half + 1) * HALF].astype(
                        jnp.bfloat16),
                    preferred_element_type=jnp.float32,
                )
                h_ref[half] = p_half.astype(jnp.bfloat16)
                for mask in MASKS_DESC:
                    j = jnp.bitwise_xor(my_p, mask)
                    c = jnp.bitwise_xor(my_q, L[mask])
                    rs_send(layer, half, j, c)

            for half in range(2):
                acc = h_ref[half, pl.ds(my_row, CHUNK), :].astype(jnp.float32)
                for mask in MASKS_ASC:
                    j = jnp.bitwise_xor(my_p, mask)
                    rs_desc(layer, half, j).wait_recv()
                    acc = acc + rs_rbuf[half, j].astype(jnp.float32)
                myh = jnp.maximum(acc, 0.0).astype(jnp.bfloat16)
                if layer > 0:
                    for mask in MASKS_ASC:
                        j = jnp.bitwise_xor(my_p, mask)
                        ag_desc(layer - 1, half, j, 0).wait_send()
                hg_ref[half, pl.ds(my_row, CHUNK), :] = myh
                for mask in MASKS_DESC:
                    ag_send(layer, half, jnp.bitwise_xor(my_p, mask))

            xn = None
            for half in range(2):
                for mask in MASKS_ASC:
                    j = jnp.bitwise_xor(my_p, mask)
                    c = jnp.bitwise_xor(my_q, L[mask])
                    ag_desc(layer, half, j, c).wait_recv()
                contrib = jnp.dot(
                    hg_ref[half],
                    w_out[half * HALF:(half + 1) * HALF, :].astype(
                        jnp.bfloat16),
                    preferred_element_type=jnp.float32,
                )
                xn = contrib if xn is None else xn + contrib

            if layer < N_LAYERS - 1:
                xv = xn.astype(jnp.bfloat16)
            else:
                out_ref[...] = xn

            for half in range(2):
                for mask in MASKS_ASC:
                    j = jnp.bitwise_xor(my_p, mask)
                    rs_desc(layer, half, j).wait_send()

        for half in range(2):
            for mask in MASKS_ASC:
                j = jnp.bitwise_xor(my_p, mask)
                ag_desc(N_LAYERS - 1, half, j, 0).wait_send()

    n_sems = N_LAYERS * 2 * N_DEV
    return pl.pallas_call(
        body,
        out_shape=jax.ShapeDtypeStruct((b, o_sh), jnp.float32),
        in_specs=[pl.BlockSpec(memory_space=pltpu.VMEM)] * 7,
        out_specs=pl.BlockSpec(memory_space=pltpu.VMEM),
        scratch_shapes=[
            pltpu.VMEM((2, b, HALF), jnp.bfloat16),
            pltpu.VMEM((2, N_DEV, CHUNK, HALF), jnp.bfloat16),
            pltpu.VMEM((2, b, HALF), jnp.bfloat16),
            pltpu.SemaphoreType.DMA((n_sems,)),
            pltpu.SemaphoreType.DMA((n_sems,)),
            pltpu.SemaphoreType.DMA((n_sems,)),
            pltpu.SemaphoreType.DMA((n_sems,)),
        ],
    )(x, Win0, Wout0, Win1, Wout1, Win2, Wout2)
